# Initial kernel scaffold; baseline (speedup 1.0000x reference)
import jax
import jax.numpy as jnp
from jax.experimental import pallas as pl


def kernel(ids, weight):
    raise NotImplementedError("write your pallas kernel here")


# speedup vs baseline: 1.8070x; 1.8070x over previous
import functools

import jax
import jax.numpy as jnp
from jax import lax
from jax.experimental import pallas as pl
from jax.experimental.pallas import tpu as pltpu
from jax.experimental.pallas import tpu_sc as plsc

_NUM_CORES = 2
_NUM_SUBCORES = 16
_NUM_WORKERS = _NUM_CORES * _NUM_SUBCORES
_CHUNK = 512


def _emb_lookup(flat_ids, weight, n, d):
    per_w = n // _NUM_WORKERS
    niter = per_w // _CHUNK
    mesh = plsc.VectorSubcoreMesh(core_axis_name="c", subcore_axis_name="s")

    @functools.partial(
        pl.kernel,
        out_type=jax.ShapeDtypeStruct((n, d), jnp.float32),
        mesh=mesh,
        scratch_types=[
            pltpu.VMEM((_CHUNK,), jnp.int32),
            pltpu.VMEM((_CHUNK, d), jnp.float32),
            pltpu.SemaphoreType.DMA,
        ],
        compiler_params=pltpu.CompilerParams(use_tc_tiling_on_sc=False),
    )
    def emb(ids_hbm, w_hbm, out_hbm, idx_v, rows_v, sem):
        wid = lax.axis_index("s") * _NUM_CORES + lax.axis_index("c")
        base = wid * per_w

        @pl.loop(0, niter)
        def _(i):
            off = base + i * _CHUNK
            pltpu.sync_copy(ids_hbm.at[pl.ds(off, _CHUNK)], idx_v)
            pltpu.async_copy(w_hbm.at[idx_v], rows_v, sem).wait()
            pltpu.sync_copy(rows_v, out_hbm.at[pl.ds(off, _CHUNK)])

    return emb(flat_ids, weight)


def kernel(ids, weight):
    b, h = ids.shape
    v, d = weight.shape
    n = b * h
    out = _emb_lookup(ids.reshape(n).astype(jnp.int32), weight, n, d)
    return out.reshape(b, h, d)


# speedup vs baseline: 1.8693x; 1.0345x over previous
import functools

import jax
import jax.numpy as jnp
from jax import lax
from jax.experimental import pallas as pl
from jax.experimental.pallas import tpu as pltpu
from jax.experimental.pallas import tpu_sc as plsc

_NUM_CORES = 2
_NUM_SUBCORES = 16
_NUM_WORKERS = _NUM_CORES * _NUM_SUBCORES
_CHUNK = 256
_NBUF = 4


def _emb_lookup(flat_ids, weight, n, d):
    per_w = n // _NUM_WORKERS
    niter = per_w // _CHUNK
    assert niter % _NBUF == 0
    mesh = plsc.VectorSubcoreMesh(core_axis_name="c", subcore_axis_name="s")

    @functools.partial(
        pl.kernel,
        out_type=jax.ShapeDtypeStruct((n, d), jnp.float32),
        mesh=mesh,
        scratch_types=[
            pltpu.VMEM((niter, _CHUNK), jnp.int32),
            pltpu.VMEM((_NBUF, _CHUNK, d), jnp.float32),
            pltpu.SemaphoreType.DMA((_NBUF,)),
            pltpu.SemaphoreType.DMA((_NBUF,)),
        ],
        compiler_params=pltpu.CompilerParams(use_tc_tiling_on_sc=False),
    )
    def emb(ids_hbm, w_hbm, out_hbm, idx_v, rows_v, gsem, ssem):
        wid = lax.axis_index("s") * _NUM_CORES + lax.axis_index("c")
        base = wid * per_w

        pltpu.sync_copy(ids_hbm.at[wid], idx_v)

        def start_gather(i, b):
            pltpu.async_copy(w_hbm.at[idx_v.at[i]], rows_v.at[b], gsem.at[b])

        def wait_gather(b):
            pltpu.make_async_copy(
                w_hbm.at[idx_v.at[0]], rows_v.at[b], gsem.at[b]
            ).wait()

        def start_store(i, b):
            pltpu.async_copy(
                rows_v.at[b], out_hbm.at[pl.ds(base + i * _CHUNK, _CHUNK)],
                ssem.at[b],
            )

        def wait_store(i, b):
            pltpu.make_async_copy(
                rows_v.at[b], out_hbm.at[pl.ds(base + i * _CHUNK, _CHUNK)],
                ssem.at[b],
            ).wait()

        for b in range(_NBUF):
            start_gather(b, b)

        @pl.loop(0, niter, step=_NBUF)
        def _(g):
            for b in range(_NBUF):
                wait_gather(b)
                start_store(g + b, b)
            for b in range(_NBUF):
                @pl.when(g + b + _NBUF < niter)
                def _():
                    wait_store(g + b, b)
                    start_gather(g + b + _NBUF, b)

        for b in range(_NBUF):
            wait_store(niter - _NBUF + b, b)

    ids3d = flat_ids.reshape(_NUM_WORKERS, niter, _CHUNK)
    return emb(ids3d, weight)


def kernel(ids, weight):
    b, h = ids.shape
    v, d = weight.shape
    n = b * h
    out = _emb_lookup(ids.reshape(n).astype(jnp.int32), weight, n, d)
    return out.reshape(b, h, d)
